# Initial kernel scaffold; baseline (speedup 1.0000x reference)
import jax
import jax.numpy as jnp
from jax.experimental import pallas as pl


def kernel(encoder_input, decoder_input, labels, W_enc, emb, W_dec):
    raise NotImplementedError("write your pallas kernel here")


# speedup vs baseline: 8.1176x; 8.1176x over previous
import jax
import jax.numpy as jnp
from jax.experimental import pallas as pl
from jax.experimental.pallas import tpu as pltpu

_B, _V, _D, _K = 64, 4096, 1024, 8
_BV = 512
_NBLK = _V // _BV


def _fused_body(enc, dec, lab, wenc, emb, wdec, gum, out,
                scores_s, g_s, e2_s, r_s):
    j = pl.program_id(0)

    @pl.when(j == 0)
    def _init():
        r_s[...] = jnp.dot(dec[...], wdec[...],
                           preferred_element_type=jnp.float32) - lab[...]

    scores_s[:, pl.ds(j * _BV, _BV)] = jnp.dot(
        enc[...], wenc[...], preferred_element_type=jnp.float32)
    eblk = emb[...]
    g_s[:, pl.ds(j * _BV, _BV)] = jax.lax.dot_general(
        r_s[...], eblk, (((1,), (1,)), ((), ())),
        preferred_element_type=jnp.float32)
    e2_s[:, pl.ds(j * _BV, _BV)] = jnp.sum(eblk * eblk, axis=1)[None, :]

    @pl.when(j == _NBLK - 1)
    def _combine():
        scores = scores_s[...]
        m = jnp.max(scores, axis=-1, keepdims=True)
        ex = jnp.exp(scores - m)
        se = jnp.sum(ex, axis=-1, keepdims=True)
        prob = ex / se
        logp = scores - m - jnp.log(se)

        r = r_s[...]
        r2 = jnp.sum(r * r, axis=1, keepdims=True)
        loss_arr = (e2_s[...] + 2.0 * g_s[...] + r2) * (1.0 / _D)
        coef_arr = (1.0 + logp) * prob

        iota = jax.lax.broadcasted_iota(jnp.int32, (_B, _V), 1)

        def tk_body(_, carry):
            work, maskf = carry
            rowmax = jnp.max(work, axis=-1, keepdims=True)
            cand = jnp.where(work == rowmax, iota, _V)
            first = jnp.min(cand, axis=-1, keepdims=True)
            onehot = iota == first
            maskf = maskf + onehot.astype(jnp.float32)
            work = jnp.where(onehot, -1.0, work)
            return work, maskf

        _, maskf = jax.lax.fori_loop(
            0, _K, tk_body, (prob, jnp.zeros((_B, _V), jnp.float32)))

        summed = jnp.sum(maskf * coef_arr * loss_arr, axis=-1)
        sw = jnp.sum(prob * (1.0 - maskf), axis=-1, keepdims=True)
        cond = (prob + 1e-12) * (1.0 - maskf) / (sw + 1e-12)
        logits = jnp.log(cond) + gum[...]
        rowmax2 = jnp.max(logits, axis=-1, keepdims=True)
        cand2 = jnp.where(logits == rowmax2, iota, _V)
        first2 = jnp.min(cand2, axis=-1, keepdims=True)
        onehot2 = iota == first2
        loss_smp = jnp.sum(jnp.where(onehot2, loss_arr, 0.0), axis=-1)
        lp_smp = jnp.sum(jnp.where(onehot2, logp, 0.0), axis=-1)

        loss_b = loss_smp * (1.0 + lp_smp) * sw[:, 0] + summed
        out[0, 0] = jnp.mean(loss_b)


def kernel(encoder_input, decoder_input, labels, W_enc, emb, W_dec):
    gum = jax.random.gumbel(jax.random.key(123), (_B, _V), jnp.float32)
    out = pl.pallas_call(
        _fused_body,
        grid=(_NBLK,),
        in_specs=[
            pl.BlockSpec((_B, _D), lambda j: (0, 0)),
            pl.BlockSpec((_B, _D), lambda j: (0, 0)),
            pl.BlockSpec((_B, _D), lambda j: (0, 0)),
            pl.BlockSpec((_D, _BV), lambda j: (0, j)),
            pl.BlockSpec((_BV, _D), lambda j: (j, 0)),
            pl.BlockSpec((_D, _D), lambda j: (0, 0)),
            pl.BlockSpec((_B, _V), lambda j: (0, 0)),
        ],
        out_shape=jax.ShapeDtypeStruct((1, 1), jnp.float32),
        out_specs=pl.BlockSpec(memory_space=pltpu.SMEM),
        scratch_shapes=[
            pltpu.VMEM((_B, _V), jnp.float32),
            pltpu.VMEM((_B, _V), jnp.float32),
            pltpu.VMEM((1, _V), jnp.float32),
            pltpu.VMEM((_B, _D), jnp.float32),
        ],
        compiler_params=pltpu.CompilerParams(
            dimension_semantics=("arbitrary",),
        ),
    )(encoder_input, decoder_input, labels, W_enc, emb, W_dec, gum)
    return out[0, 0]
